# baseline (speedup 1.0000x reference)
import functools

import jax
import jax.numpy as jnp
from jax import lax
from jax.experimental import pallas as pl
from jax.experimental.pallas import tpu as pltpu
from jax.experimental.pallas import tpu_sc as plsc

BATCH = 16384
NUM_FIELDS = 26
VOCAB = 100000
EMBED_DIM = 16

NC = 2
NS = 16
LANES = 16
NW = NC * NS

N = BATCH * NUM_FIELDS
ROWS_PER_W = N // NW
CHUNK = 1664
NCHUNK = ROWS_PER_W // CHUNK
SLICES = CHUNK // LANES


def _body(idx_hbm, tab_hbm, out_hbm, idx_v, comb_v, rows_v, sem):
    wid = lax.axis_index("s") * NC + lax.axis_index("c")
    base = wid * ROWS_PER_W

    def chunk_step(c, _):
        start = base + c * CHUNK
        pltpu.sync_copy(idx_hbm.at[pl.ds(start, CHUNK)], idx_v)

        def slice_step(j, _):
            pos = lax.broadcasted_iota(jnp.int32, (LANES,), 0) + j * LANES
            field = lax.rem(pos, NUM_FIELDS)
            comb_v[pl.ds(j * LANES, LANES)] = (
                idx_v[pl.ds(j * LANES, LANES)] + field * VOCAB
            )
            return 0

        lax.fori_loop(0, SLICES, slice_step, 0)

        pltpu.async_copy(tab_hbm.at[comb_v], rows_v, sem).wait()
        pltpu.sync_copy(rows_v, out_hbm.at[pl.ds(start, CHUNK)])
        return 0

    lax.fori_loop(0, NCHUNK, chunk_step, 0)


@functools.lru_cache(maxsize=1)
def _gather_kernel():
    return functools.partial(
        pl.kernel,
        out_type=jax.ShapeDtypeStruct((N, EMBED_DIM), jnp.float32),
        mesh=plsc.VectorSubcoreMesh(
            core_axis_name="c", subcore_axis_name="s", num_cores=NC, num_subcores=NS
        ),
        scratch_types=[
            pltpu.VMEM((CHUNK,), jnp.int32),
            pltpu.VMEM((CHUNK,), jnp.int32),
            pltpu.VMEM((CHUNK, EMBED_DIM), jnp.float32),
            pltpu.SemaphoreType.DMA,
        ],
        compiler_params=pltpu.CompilerParams(use_tc_tiling_on_sc=False),
    )(_body)


def kernel(indices, tables):
    idx_flat = indices.astype(jnp.int32).reshape(N)
    tab_flat = tables.reshape(NUM_FIELDS * VOCAB, EMBED_DIM)
    out = _gather_kernel()(idx_flat, tab_flat)
    return out.reshape(BATCH, NUM_FIELDS * EMBED_DIM)


# speedup vs baseline: 1.0125x; 1.0125x over previous
import functools

import jax
import jax.numpy as jnp
from jax import lax
from jax.experimental import pallas as pl
from jax.experimental.pallas import tpu as pltpu
from jax.experimental.pallas import tpu_sc as plsc

BATCH = 16384
NUM_FIELDS = 26
VOCAB = 100000
EMBED_DIM = 16

NC = 2
NS = 16
LANES = 16
NW = NC * NS

N = BATCH * NUM_FIELDS
ROWS_PER_W = N // NW
CHUNK = 1664
NCHUNK = ROWS_PER_W // CHUNK
SLICES = CHUNK // LANES


def _body(idx_hbm, tab_hbm, out_hbm, idx2, comb2, offs_v, rows2,
          gsem0, gsem1, wsem0, wsem1):
    wid = lax.axis_index("s") * NC + lax.axis_index("c")
    base = wid * ROWS_PER_W
    gsems = (gsem0, gsem1)
    wsems = (wsem0, wsem1)

    def offs_step(j, _):
        pos = lax.broadcasted_iota(jnp.int32, (LANES,), 0) + j * LANES
        offs_v[pl.ds(j * LANES, LANES)] = lax.rem(pos, NUM_FIELDS) * VOCAB
        return 0

    lax.fori_loop(0, SLICES, offs_step, 0)

    def load_and_combine(c, cur):
        start = base + c * CHUNK
        pltpu.sync_copy(idx_hbm.at[pl.ds(start, CHUNK)], idx2.at[cur])
        idx_ref = idx2.at[cur]
        comb_ref = comb2.at[cur]

        def slice_step(j, _):
            comb_ref[pl.ds(j * LANES, LANES)] = (
                idx_ref[pl.ds(j * LANES, LANES)] + offs_v[pl.ds(j * LANES, LANES)]
            )
            return 0

        lax.fori_loop(0, SLICES, slice_step, 0)

    gh = [None] * NCHUNK
    wh = [None] * NCHUNK
    for c in range(NCHUNK):
        cur = c & 1
        load_and_combine(c, cur)
        if c >= 2:
            wh[c - 2].wait()
        gh[c] = pltpu.async_copy(tab_hbm.at[comb2.at[cur]], rows2.at[cur], gsems[cur])
        if c >= 1:
            prev = 1 - cur
            gh[c - 1].wait()
            wh[c - 1] = pltpu.async_copy(
                rows2.at[prev],
                out_hbm.at[pl.ds(base + (c - 1) * CHUNK, CHUNK)],
                wsems[prev],
            )
    last = NCHUNK - 1
    gh[last].wait()
    wh[last] = pltpu.async_copy(
        rows2.at[last & 1],
        out_hbm.at[pl.ds(base + last * CHUNK, CHUNK)],
        wsems[last & 1],
    )
    wh[last - 1].wait()
    wh[last].wait()


@functools.lru_cache(maxsize=1)
def _gather_kernel():
    return functools.partial(
        pl.kernel,
        out_type=jax.ShapeDtypeStruct((N, EMBED_DIM), jnp.float32),
        mesh=plsc.VectorSubcoreMesh(
            core_axis_name="c", subcore_axis_name="s", num_cores=NC, num_subcores=NS
        ),
        scratch_types=[
            pltpu.VMEM((2, CHUNK), jnp.int32),
            pltpu.VMEM((2, CHUNK), jnp.int32),
            pltpu.VMEM((CHUNK,), jnp.int32),
            pltpu.VMEM((2, CHUNK, EMBED_DIM), jnp.float32),
            pltpu.SemaphoreType.DMA,
            pltpu.SemaphoreType.DMA,
            pltpu.SemaphoreType.DMA,
            pltpu.SemaphoreType.DMA,
        ],
        compiler_params=pltpu.CompilerParams(use_tc_tiling_on_sc=False),
    )(_body)


def kernel(indices, tables):
    idx_flat = indices.astype(jnp.int32).reshape(N)
    tab_flat = tables.reshape(NUM_FIELDS * VOCAB, EMBED_DIM)
    out = _gather_kernel()(idx_flat, tab_flat)
    return out.reshape(BATCH, NUM_FIELDS * EMBED_DIM)


# speedup vs baseline: 4.6746x; 4.6171x over previous
import functools

import jax
import jax.numpy as jnp
from jax import lax
from jax.experimental import pallas as pl
from jax.experimental.pallas import tpu as pltpu
from jax.experimental.pallas import tpu_sc as plsc

BATCH = 16384
NUM_FIELDS = 26
VOCAB = 100000
EMBED_DIM = 16

NC = 2
NS = 16
LANES = 16
NW = NC * NS

R = NUM_FIELDS * EMBED_DIM
ROWS_PER_W = R // NW
BCHUNK = 4096
NBCHUNK = BATCH // BCHUNK
SLICES = BCHUNK // LANES


def _body(tab_hbm, idx_hbm, out_hbm, row_v, idx2, out2, gsem, wsem, rsem):
    wid = lax.axis_index("s") * NC + lax.axis_index("c")
    zeros16 = lax.broadcasted_iota(jnp.int32, (LANES,), 0) * 0

    def row_step(k, _):
        r = wid * ROWS_PER_W + k
        f = r // EMBED_DIM
        pltpu.async_copy(tab_hbm.at[pl.ds(r, 1), :], row_v, rsem).wait()

        def chunk_step(c, _):
            cur = lax.rem(c, 2) * BCHUNK
            b0 = c * BCHUNK
            pltpu.sync_copy(idx_hbm.at[pl.ds(f * BATCH + b0, BCHUNK)],
                            idx2.at[pl.ds(cur, BCHUNK)])

            def slice_step(j, _):
                vidx = idx2[pl.ds(cur + j * LANES, LANES)]
                out2[pl.ds(cur + j * LANES, LANES)] = plsc.load_gather(
                    row_v, [zeros16, vidx]
                )
                return 0

            lax.fori_loop(0, SLICES, slice_step, 0)
            pltpu.sync_copy(out2.at[pl.ds(cur, BCHUNK)],
                            out_hbm.at[pl.ds(r * BATCH + b0, BCHUNK)])
            return 0

        lax.fori_loop(0, NBCHUNK, chunk_step, 0)
        return 0

    lax.fori_loop(0, ROWS_PER_W, row_step, 0)


@functools.lru_cache(maxsize=1)
def _gather_kernel():
    return functools.partial(
        pl.kernel,
        out_type=jax.ShapeDtypeStruct((R * BATCH,), jnp.float32),
        mesh=plsc.VectorSubcoreMesh(
            core_axis_name="c", subcore_axis_name="s", num_cores=NC, num_subcores=NS
        ),
        scratch_types=[
            pltpu.VMEM((1, VOCAB), jnp.float32),
            pltpu.VMEM((2 * BCHUNK,), jnp.int32),
            pltpu.VMEM((2 * BCHUNK,), jnp.float32),
            pltpu.SemaphoreType.DMA,
            pltpu.SemaphoreType.DMA,
            pltpu.SemaphoreType.DMA,
        ],
        compiler_params=pltpu.CompilerParams(
            use_tc_tiling_on_sc=True, needs_layout_passes=False
        ),
    )(_body)


def kernel(indices, tables):
    tab2d = jnp.transpose(tables, (0, 2, 1)).reshape(R, VOCAB)
    idx_lin = jnp.transpose(indices.astype(jnp.int32), (1, 0)).reshape(
        NUM_FIELDS * BATCH
    )
    out_lin = _gather_kernel()(tab2d, idx_lin)
    return jnp.transpose(out_lin.reshape(R, BATCH), (1, 0))


# speedup vs baseline: 5.3292x; 1.1400x over previous
import functools

import jax
import jax.numpy as jnp
from jax import lax
from jax.experimental import pallas as pl
from jax.experimental.pallas import tpu as pltpu
from jax.experimental.pallas import tpu_sc as plsc

BATCH = 16384
NUM_FIELDS = 26
VOCAB = 100000
EMBED_DIM = 16

NC = 2
NS = 16
LANES = 16
NW = NC * NS

R = NUM_FIELDS * EMBED_DIM
ROWS_PER_W = R // NW
BCHUNK = 4096
NBCHUNK = BATCH // BCHUNK
NCH = ROWS_PER_W * NBCHUNK
SLICES = BCHUNK // LANES
UNROLL = 8


def _body(tab_hbm, idx_hbm, out_hbm, row_v, idx2, out2, rsem, isem, wsem):
    wid = lax.axis_index("s") * NC + lax.axis_index("c")
    r0 = wid * ROWS_PER_W
    zeros16 = lax.broadcasted_iota(jnp.int32, (LANES,), 0) * 0

    def idx_copy(t, cur):
        k, c = divmod(t, NBCHUNK)
        f = (r0 + k) // EMBED_DIM
        return pltpu.async_copy(
            idx_hbm.at[pl.ds(f * BATCH + c * BCHUNK, BCHUNK)],
            idx2.at[pl.ds(cur * BCHUNK, BCHUNK)],
            isem,
        )

    def row_copy(k):
        return pltpu.async_copy(tab_hbm.at[pl.ds(r0 + k, 1), :], row_v, rsem)

    ih = idx_copy(0, 0)
    rh = row_copy(0)
    rh.wait()
    wh = [None] * NCH
    for t in range(NCH):
        k, c = divmod(t, NBCHUNK)
        cur = t & 1
        ih.wait()
        if t + 1 < NCH:
            ih = idx_copy(t + 1, 1 - cur)
        if t >= 2:
            wh[t - 2].wait()

        def slice_step(j, _, cur=cur):
            base = cur * BCHUNK + j * (LANES * UNROLL)
            for u in range(UNROLL):
                off = base + u * LANES
                vidx = idx2[pl.ds(off, LANES)]
                out2[cur, pl.ds(off - cur * BCHUNK, LANES)] = plsc.load_gather(
                    row_v, [zeros16, vidx]
                )
            return 0

        lax.fori_loop(0, SLICES // UNROLL, slice_step, 0)

        if c == NBCHUNK - 1 and k + 1 < ROWS_PER_W:
            rh = row_copy(k + 1)
        wh[t] = pltpu.async_copy(
            out2.at[pl.ds(cur, 1), :],
            out_hbm.at[pl.ds(r0 + k, 1), pl.ds(c * BCHUNK, BCHUNK)],
            wsem,
        )
        if c == NBCHUNK - 1 and k + 1 < ROWS_PER_W:
            rh.wait()
    wh[NCH - 2].wait()
    wh[NCH - 1].wait()


@functools.lru_cache(maxsize=1)
def _gather_kernel():
    return functools.partial(
        pl.kernel,
        out_type=jax.ShapeDtypeStruct((R, BATCH), jnp.float32),
        mesh=plsc.VectorSubcoreMesh(
            core_axis_name="c", subcore_axis_name="s", num_cores=NC, num_subcores=NS
        ),
        scratch_types=[
            pltpu.VMEM((1, VOCAB), jnp.float32),
            pltpu.VMEM((2 * BCHUNK,), jnp.int32),
            pltpu.VMEM((2, BCHUNK), jnp.float32),
            pltpu.SemaphoreType.DMA,
            pltpu.SemaphoreType.DMA,
            pltpu.SemaphoreType.DMA,
        ],
        compiler_params=pltpu.CompilerParams(
            use_tc_tiling_on_sc=True, needs_layout_passes=False
        ),
    )(_body)


def kernel(indices, tables):
    tab2d = jnp.transpose(tables, (0, 2, 1)).reshape(R, VOCAB)
    idx_lin = jnp.transpose(indices.astype(jnp.int32), (1, 0)).reshape(
        NUM_FIELDS * BATCH
    )
    out_t = _gather_kernel()(tab2d, idx_lin)
    return jnp.transpose(out_t, (1, 0))
